# baseline (speedup 1.0000x reference)
import functools

import jax
import jax.numpy as jnp
from jax import lax
from jax.experimental import pallas as pl
from jax.experimental.pallas import tpu as pltpu
from jax.experimental.pallas import tpu_sc as plsc

B, N, D, H, W, C = 4, 6, 41, 16, 44, 64
NXY = 200
VOX_B = NXY * NXY
P_TOTAL = B * N * D * H * W
P_BATCH = P_TOTAL // B
TRASH = VOX_B

NTILES = 16
GROUPS = N * D
CH = C // 2
ZROWS_TILE = 2504
ACC_ROWS = NTILES * ZROWS_TILE
WB_TILE = VOX_B // NTILES
NSLAB = P_TOTAL // W

VROWS = P_TOTAL // 128
VBLK = 512


def _vox_body(gx_ref, gy_ref, gz_ref, o_ref):
    gx = gx_ref[...]
    gy = gy_ref[...]
    gz = gz_ref[...]
    ix = jnp.floor((gx + 50.0) / 0.5).astype(jnp.int32)
    iy = jnp.floor((gy + 50.0) / 0.5).astype(jnp.int32)
    iz = jnp.floor((gz + 10.0) / 20.0).astype(jnp.int32)
    kept = (
        (ix >= 0) & (ix < NXY)
        & (iy >= 0) & (iy < NXY)
        & (iz >= 0) & (iz < 1)
    )
    o_ref[...] = jnp.where(kept, ix * NXY + iy, TRASH)


def _voxelize(gx2, gy2, gz2):
    return pl.pallas_call(
        _vox_body,
        grid=(pl.cdiv(VROWS, VBLK),),
        in_specs=[pl.BlockSpec((VBLK, 128), lambda i: (i, 0))] * 3,
        out_specs=pl.BlockSpec((VBLK, 128), lambda i: (i, 0)),
        out_shape=jax.ShapeDtypeStruct((VROWS, 128), jnp.int32),
    )(gx2, gy2, gz2)


KMAX = (GROUPS + NTILES - 1) // NTILES


def _sc_scatter_body(b0, vloc_hbm, x_hbm, out_hbm,
                     idx_v, rows_v, zbuf, acc, sem_ld, sem_sc, sem_z):
    c = lax.axis_index("c")
    s = lax.axis_index("s")
    col0 = c * CH

    def zrow(i, carry):
        zbuf[i, pl.ds(0, 16)] = jnp.zeros((16,), jnp.float32)
        zbuf[i, pl.ds(16, 16)] = jnp.zeros((16,), jnp.float32)
        return carry
    lax.fori_loop(0, 128, zrow, 0)

    def issue_loads(b, k, buf):
        gp = s + NTILES * k

        @pl.when(gp < GROUPS)
        def _():
            n = gp // D
            d = gp % D
            gg = (b0 + b) * GROUPS + gp
            pltpu.async_copy(
                vloc_hbm.at[pl.ds(H * gg, H), :], idx_v.at[buf], sem_ld.at[buf])
            pltpu.async_copy(
                x_hbm.at[b, n, d, :, :, pl.ds(col0, CH)], rows_v.at[buf],
                sem_ld.at[buf])

    def wait_loads(buf):
        pltpu.make_async_copy(
            vloc_hbm.at[pl.ds(0, H), :], idx_v.at[buf], sem_ld.at[buf]).wait()
        pltpu.make_async_copy(
            x_hbm.at[0, 0, 0, :, :, pl.ds(col0, CH)], rows_v.at[buf],
            sem_ld.at[buf]).wait()

    def batch(b, carry):
        for k in range(19):
            pltpu.async_copy(
                zbuf, acc.at[pl.ds(s * ZROWS_TILE + k * 128, 128), :], sem_z)
        pltpu.async_copy(
            zbuf.at[pl.ds(0, 72), :],
            acc.at[pl.ds(s * ZROWS_TILE + 2432, 72), :], sem_z)
        for k in range(19):
            pltpu.make_async_copy(
                zbuf, acc.at[pl.ds(0, 128), :], sem_z).wait()
        pltpu.make_async_copy(
            zbuf.at[pl.ds(0, 72), :], acc.at[pl.ds(0, 72), :], sem_z).wait()
        plsc.subcore_barrier()

        issue_loads(b, 0, 0)
        for k in range(KMAX):
            cb = k % 2
            gp = s + NTILES * k
            if k + 1 < KMAX:
                issue_loads(b, k + 1, 1 - cb)

            @pl.when(gp < GROUPS)
            def _():
                wait_loads(cb)
                for j in range(H):
                    pltpu.async_copy(
                        rows_v.at[cb, j], acc.at[idx_v.at[cb, j]], sem_sc,
                        add=True)
                for j in range(H):
                    pltpu.make_async_copy(
                        rows_v.at[cb, 0], acc.at[pl.ds(0, W), :],
                        sem_sc).wait()
        plsc.subcore_barrier()

        pltpu.sync_copy(
            acc.at[pl.ds(s * WB_TILE, WB_TILE), :],
            out_hbm.at[pl.ds(b * VOX_B + s * WB_TILE, WB_TILE),
                       pl.ds(col0, CH)])
        plsc.subcore_barrier()
        return carry

    lax.fori_loop(0, x_hbm.shape[0], batch, 0)


def _sc_scatter_half(b0, vloc44, xhalf):
    nb = xhalf.shape[0]
    call = functools.partial(
        pl.kernel,
        out_type=jax.ShapeDtypeStruct((nb * VOX_B, C), jnp.float32),
        mesh=plsc.VectorSubcoreMesh(core_axis_name="c", subcore_axis_name="s"),
        compiler_params=pltpu.CompilerParams(use_tc_tiling_on_sc=False),
        scratch_types=[
            pltpu.VMEM((2, H, W), jnp.int32),
            pltpu.VMEM((2, H, W, CH), jnp.float32),
            pltpu.VMEM((128, CH), jnp.float32),
            pltpu.VMEM_SHARED((ACC_ROWS, CH), jnp.float32),
            pltpu.SemaphoreType.DMA((2,)),
            pltpu.SemaphoreType.DMA,
            pltpu.SemaphoreType.DMA,
        ],
    )(functools.partial(_sc_scatter_body, b0))
    return call(vloc44, xhalf)


def _tr_body(i_ref, o_ref):
    o_ref[0] = i_ref[0].T


def _transpose(summed, nb):
    TR = 2048
    return pl.pallas_call(
        _tr_body,
        grid=(nb, pl.cdiv(VOX_B, TR)),
        in_specs=[pl.BlockSpec((1, TR, C), lambda b, r: (b, r, 0))],
        out_specs=pl.BlockSpec((1, C, TR), lambda b, r: (b, 0, r)),
        out_shape=jax.ShapeDtypeStruct((nb, C, VOX_B), jnp.float32),
    )(summed.reshape(nb, VOX_B, C))


def kernel(x, geom):
    gx2 = geom[..., 0].reshape(VROWS, 128)
    gy2 = geom[..., 1].reshape(VROWS, 128)
    gz2 = geom[..., 2].reshape(VROWS, 128)
    vloc44 = _voxelize(gx2, gy2, gz2).reshape(NSLAB, W)
    t_halves = []
    for b0 in (0, 2):
        summed = _sc_scatter_half(b0, vloc44, lax.slice_in_dim(x, b0, b0 + 2))
        t_halves.append(_transpose(summed, 2))
    out = jnp.concatenate(t_halves, axis=0)
    return out.reshape(B, C, NXY, NXY)


# speedup vs baseline: 1.2424x; 1.2424x over previous
import functools

import jax
import jax.numpy as jnp
from jax import lax
from jax.experimental import pallas as pl
from jax.experimental.pallas import tpu as pltpu
from jax.experimental.pallas import tpu_sc as plsc

B, N, D, H, W, C = 4, 6, 41, 16, 44, 64
NXY = 200
VOX_B = NXY * NXY
P_TOTAL = B * N * D * H * W
P_BATCH = P_TOTAL // B
TRASH = VOX_B

NTILES = 16
GROUPS = N * D
CH = C // 2
ZROWS_TILE = 2504
ACC_ROWS = NTILES * ZROWS_TILE
WB_TILE = VOX_B // NTILES
NSLAB = P_TOTAL // W

VROWS = P_TOTAL // 128
VBLK = 512


def _vox_body(gx_ref, gy_ref, gz_ref, o_ref):
    gx = gx_ref[...]
    gy = gy_ref[...]
    gz = gz_ref[...]
    ix = jnp.floor((gx + 50.0) / 0.5).astype(jnp.int32)
    iy = jnp.floor((gy + 50.0) / 0.5).astype(jnp.int32)
    iz = jnp.floor((gz + 10.0) / 20.0).astype(jnp.int32)
    kept = (
        (ix >= 0) & (ix < NXY)
        & (iy >= 0) & (iy < NXY)
        & (iz >= 0) & (iz < 1)
    )
    o_ref[...] = jnp.where(kept, ix * NXY + iy, TRASH)


def _voxelize(gx2, gy2, gz2):
    return pl.pallas_call(
        _vox_body,
        grid=(pl.cdiv(VROWS, VBLK),),
        in_specs=[pl.BlockSpec((VBLK, 128), lambda i: (i, 0))] * 3,
        out_specs=pl.BlockSpec((VBLK, 128), lambda i: (i, 0)),
        out_shape=jax.ShapeDtypeStruct((VROWS, 128), jnp.int32),
    )(gx2, gy2, gz2)


KMAX = (GROUPS + NTILES - 1) // NTILES


def _sc_scatter_body(b0, vloc_hbm, x_hbm, out_hbm,
                     idx_v, rows_v, zbuf, acc, sem_ld, sem_sc, sem_z):
    c = lax.axis_index("c")
    s = lax.axis_index("s")
    col0 = c * CH

    def zrow(i, carry):
        zbuf[i, pl.ds(0, 16)] = jnp.zeros((16,), jnp.float32)
        zbuf[i, pl.ds(16, 16)] = jnp.zeros((16,), jnp.float32)
        return carry
    lax.fori_loop(0, 128, zrow, 0)

    def issue_loads(b, k, buf):
        gp = s + NTILES * k

        @pl.when(gp < GROUPS)
        def _():
            n = gp // D
            d = gp % D
            gg = (b0 + b) * GROUPS + gp
            pltpu.async_copy(
                vloc_hbm.at[pl.ds(H * gg, H), :], idx_v.at[buf], sem_ld.at[buf])
            pltpu.async_copy(
                x_hbm.at[b, n, d, :, :, pl.ds(col0, CH)], rows_v.at[buf],
                sem_ld.at[buf])

    def wait_loads(buf):
        pltpu.make_async_copy(
            vloc_hbm.at[pl.ds(0, H), :], idx_v.at[buf], sem_ld.at[buf]).wait()
        pltpu.make_async_copy(
            x_hbm.at[0, 0, 0, :, :, pl.ds(col0, CH)], rows_v.at[buf],
            sem_ld.at[buf]).wait()

    def batch(b, carry):
        for k in range(19):
            pltpu.async_copy(
                zbuf, acc.at[pl.ds(s * ZROWS_TILE + k * 128, 128), :], sem_z)
        pltpu.async_copy(
            zbuf.at[pl.ds(0, 72), :],
            acc.at[pl.ds(s * ZROWS_TILE + 2432, 72), :], sem_z)
        for k in range(19):
            pltpu.make_async_copy(
                zbuf, acc.at[pl.ds(0, 128), :], sem_z).wait()
        pltpu.make_async_copy(
            zbuf.at[pl.ds(0, 72), :], acc.at[pl.ds(0, 72), :], sem_z).wait()
        plsc.subcore_barrier()

        issue_loads(b, 0, 0)
        for k in range(KMAX):
            cb = k % 2
            gp = s + NTILES * k
            if k >= 1:
                @pl.when(s + NTILES * (k - 1) < GROUPS)
                def _():
                    for j in range(H):
                        pltpu.make_async_copy(
                            rows_v.at[1 - cb, 0], acc.at[pl.ds(0, W), :],
                            sem_sc).wait()
            if k + 1 < KMAX:
                issue_loads(b, k + 1, 1 - cb)

            @pl.when(gp < GROUPS)
            def _():
                wait_loads(cb)
                for j in range(H):
                    pltpu.async_copy(
                        rows_v.at[cb, j], acc.at[idx_v.at[cb, j]], sem_sc,
                        add=True)

        @pl.when(s + NTILES * (KMAX - 1) < GROUPS)
        def _():
            for j in range(H):
                pltpu.make_async_copy(
                    rows_v.at[(KMAX - 1) % 2, 0], acc.at[pl.ds(0, W), :],
                    sem_sc).wait()
        plsc.subcore_barrier()

        pltpu.sync_copy(
            acc.at[pl.ds(s * WB_TILE, WB_TILE), :],
            out_hbm.at[pl.ds(b * VOX_B + s * WB_TILE, WB_TILE),
                       pl.ds(col0, CH)])
        plsc.subcore_barrier()
        return carry

    lax.fori_loop(0, x_hbm.shape[0], batch, 0)


def _sc_scatter_half(b0, vloc44, xhalf):
    nb = xhalf.shape[0]
    call = functools.partial(
        pl.kernel,
        out_type=jax.ShapeDtypeStruct((nb * VOX_B, C), jnp.float32),
        mesh=plsc.VectorSubcoreMesh(core_axis_name="c", subcore_axis_name="s"),
        compiler_params=pltpu.CompilerParams(use_tc_tiling_on_sc=False),
        scratch_types=[
            pltpu.VMEM((2, H, W), jnp.int32),
            pltpu.VMEM((2, H, W, CH), jnp.float32),
            pltpu.VMEM((128, CH), jnp.float32),
            pltpu.VMEM_SHARED((ACC_ROWS, CH), jnp.float32),
            pltpu.SemaphoreType.DMA((2,)),
            pltpu.SemaphoreType.DMA,
            pltpu.SemaphoreType.DMA,
        ],
    )(functools.partial(_sc_scatter_body, b0))
    return call(vloc44, xhalf)


def _tr_body(i_ref, o_ref):
    o_ref[0] = i_ref[0].T


def _transpose(summed, nb):
    TR = 2048
    return pl.pallas_call(
        _tr_body,
        grid=(nb, pl.cdiv(VOX_B, TR)),
        in_specs=[pl.BlockSpec((1, TR, C), lambda b, r: (b, r, 0))],
        out_specs=pl.BlockSpec((1, C, TR), lambda b, r: (b, 0, r)),
        out_shape=jax.ShapeDtypeStruct((nb, C, VOX_B), jnp.float32),
    )(summed.reshape(nb, VOX_B, C))


def kernel(x, geom):
    gx2 = geom[..., 0].reshape(VROWS, 128)
    gy2 = geom[..., 1].reshape(VROWS, 128)
    gz2 = geom[..., 2].reshape(VROWS, 128)
    vloc44 = _voxelize(gx2, gy2, gz2).reshape(NSLAB, W)
    summed = _sc_scatter_half(0, vloc44, x)
    return _transpose(summed, B).reshape(B, C, NXY, NXY)


# speedup vs baseline: 1.3473x; 1.0844x over previous
import functools

import jax
import jax.numpy as jnp
from jax import lax
from jax.experimental import pallas as pl
from jax.experimental.pallas import tpu as pltpu
from jax.experimental.pallas import tpu_sc as plsc

B, N, D, H, W, C = 4, 6, 41, 16, 44, 64
NXY = 200
VOX_B = NXY * NXY
P_TOTAL = B * N * D * H * W
P_BATCH = P_TOTAL // B
TRASH = VOX_B

NTILES = 16
GROUPS = N * D
CH = C // 2
ZROWS_TILE = 2504
ACC_ROWS = NTILES * ZROWS_TILE
WB_TILE = VOX_B // NTILES
NROWS32 = P_TOTAL // 32

VROWS = P_TOTAL // 128
VBLK = 512


def _vox_body(gx_ref, gy_ref, gz_ref, o_ref):
    gx = gx_ref[...]
    gy = gy_ref[...]
    gz = gz_ref[...]
    ix = jnp.floor((gx + 50.0) / 0.5).astype(jnp.int32)
    iy = jnp.floor((gy + 50.0) / 0.5).astype(jnp.int32)
    iz = jnp.floor((gz + 10.0) / 20.0).astype(jnp.int32)
    kept = (
        (ix >= 0) & (ix < NXY)
        & (iy >= 0) & (iy < NXY)
        & (iz >= 0) & (iz < 1)
    )
    o_ref[...] = jnp.where(kept, ix * NXY + iy, TRASH)


def _voxelize(gx2, gy2, gz2):
    return pl.pallas_call(
        _vox_body,
        grid=(pl.cdiv(VROWS, VBLK),),
        in_specs=[pl.BlockSpec((VBLK, 128), lambda i: (i, 0))] * 3,
        out_specs=pl.BlockSpec((VBLK, 128), lambda i: (i, 0)),
        out_shape=jax.ShapeDtypeStruct((VROWS, 128), jnp.int32),
    )(gx2, gy2, gz2)


KMAX = (GROUPS + NTILES - 1) // NTILES


def _sc_scatter_body(b0, vloc_hbm, x_hbm, out_hbm,
                     idx_v, rows_v, zbuf, acc, sem_ld, sem_sc, sem_z):
    c = lax.axis_index("c")
    s = lax.axis_index("s")
    col0 = c * CH

    def zrow(i, carry):
        zbuf[i, pl.ds(0, 16)] = jnp.zeros((16,), jnp.float32)
        zbuf[i, pl.ds(16, 16)] = jnp.zeros((16,), jnp.float32)
        return carry
    lax.fori_loop(0, 128, zrow, 0)

    def issue_loads(b, k, buf):
        gp = s + NTILES * k

        @pl.when(gp < GROUPS)
        def _():
            gg = (b0 + b) * GROUPS + gp
            pltpu.async_copy(
                vloc_hbm.at[pl.ds(22 * gg, 22), :], idx_v.at[buf],
                sem_ld.at[buf])
            pltpu.async_copy(
                x_hbm.at[pl.ds(704 * gg, 704), pl.ds(col0, CH)],
                rows_v.at[buf], sem_ld.at[buf])

    def wait_loads(buf):
        pltpu.make_async_copy(
            vloc_hbm.at[pl.ds(0, 22), :], idx_v.at[buf], sem_ld.at[buf]).wait()
        pltpu.make_async_copy(
            x_hbm.at[pl.ds(0, 704), pl.ds(col0, CH)], rows_v.at[buf],
            sem_ld.at[buf]).wait()

    def batch(b, carry):
        for k in range(19):
            pltpu.async_copy(
                zbuf, acc.at[pl.ds(s * ZROWS_TILE + k * 128, 128), :], sem_z)
        pltpu.async_copy(
            zbuf.at[pl.ds(0, 72), :],
            acc.at[pl.ds(s * ZROWS_TILE + 2432, 72), :], sem_z)
        for k in range(19):
            pltpu.make_async_copy(
                zbuf, acc.at[pl.ds(0, 128), :], sem_z).wait()
        pltpu.make_async_copy(
            zbuf.at[pl.ds(0, 72), :], acc.at[pl.ds(0, 72), :], sem_z).wait()
        plsc.subcore_barrier()

        issue_loads(b, 0, 0)
        for k in range(KMAX):
            cb = k % 2
            gp = s + NTILES * k
            if k >= 1:
                @pl.when(s + NTILES * (k - 1) < GROUPS)
                def _():
                    for j in range(22):
                        pltpu.make_async_copy(
                            rows_v.at[1 - cb, pl.ds(0, 32)],
                            acc.at[pl.ds(0, 32), :], sem_sc).wait()
            if k + 1 < KMAX:
                issue_loads(b, k + 1, 1 - cb)

            @pl.when(gp < GROUPS)
            def _():
                wait_loads(cb)
                for j in range(22):
                    pltpu.async_copy(
                        rows_v.at[cb, pl.ds(32 * j, 32)],
                        acc.at[idx_v.at[cb, j]], sem_sc, add=True)

        @pl.when(s + NTILES * (KMAX - 1) < GROUPS)
        def _():
            for j in range(22):
                pltpu.make_async_copy(
                    rows_v.at[(KMAX - 1) % 2, pl.ds(0, 32)],
                    acc.at[pl.ds(0, 32), :], sem_sc).wait()
        plsc.subcore_barrier()

        pltpu.sync_copy(
            acc.at[pl.ds(s * WB_TILE, WB_TILE), :],
            out_hbm.at[pl.ds(b * VOX_B + s * WB_TILE, WB_TILE),
                       pl.ds(col0, CH)])
        plsc.subcore_barrier()
        return carry

    lax.fori_loop(0, B, batch, 0)


def _sc_scatter_half(b0, vloc44, xhalf):
    nb = B
    call = functools.partial(
        pl.kernel,
        out_type=jax.ShapeDtypeStruct((nb * VOX_B, C), jnp.float32),
        mesh=plsc.VectorSubcoreMesh(core_axis_name="c", subcore_axis_name="s"),
        compiler_params=pltpu.CompilerParams(use_tc_tiling_on_sc=False),
        scratch_types=[
            pltpu.VMEM((2, 22, 32), jnp.int32),
            pltpu.VMEM((2, 704, CH), jnp.float32),
            pltpu.VMEM((128, CH), jnp.float32),
            pltpu.VMEM_SHARED((ACC_ROWS, CH), jnp.float32),
            pltpu.SemaphoreType.DMA((2,)),
            pltpu.SemaphoreType.DMA,
            pltpu.SemaphoreType.DMA,
        ],
    )(functools.partial(_sc_scatter_body, b0))
    return call(vloc44, xhalf)


def _tr_body(i_ref, o_ref):
    o_ref[0] = i_ref[0].T


def _transpose(summed, nb):
    TR = 2048
    return pl.pallas_call(
        _tr_body,
        grid=(nb, pl.cdiv(VOX_B, TR)),
        in_specs=[pl.BlockSpec((1, TR, C), lambda b, r: (b, r, 0))],
        out_specs=pl.BlockSpec((1, C, TR), lambda b, r: (b, 0, r)),
        out_shape=jax.ShapeDtypeStruct((nb, C, VOX_B), jnp.float32),
    )(summed.reshape(nb, VOX_B, C))


def kernel(x, geom):
    xp = jnp.transpose(x, (0, 1, 2, 4, 3, 5)).reshape(P_TOTAL, C)
    gp = jnp.transpose(geom, (0, 1, 2, 4, 3, 5))
    gx2 = gp[..., 0].reshape(VROWS, 128)
    gy2 = gp[..., 1].reshape(VROWS, 128)
    gz2 = gp[..., 2].reshape(VROWS, 128)
    vloc32 = _voxelize(gx2, gy2, gz2).reshape(NROWS32, 32)
    summed = _sc_scatter_half(0, vloc32, xp)
    return _transpose(summed, B).reshape(B, C, NXY, NXY)


# speedup vs baseline: 1.5430x; 1.1453x over previous
import functools

import jax
import jax.numpy as jnp
from jax import lax
from jax.experimental import pallas as pl
from jax.experimental.pallas import tpu as pltpu
from jax.experimental.pallas import tpu_sc as plsc

B, N, D, H, W, C = 4, 6, 41, 16, 44, 64
NXY = 200
VOX_B = NXY * NXY
P_TOTAL = B * N * D * H * W
P_BATCH = P_TOTAL // B
TRASH = VOX_B

NTILES = 16
GROUPS = N * D
CH = C // 2
ZROWS_TILE = 2504
ACC_ROWS = NTILES * ZROWS_TILE
WB_TILE = VOX_B // NTILES
NROWS32 = P_TOTAL // 32

VROWS = P_TOTAL // 128
VBLK = 512


def _vox_body(gx_ref, gy_ref, gz_ref, o_ref):
    gx = gx_ref[...]
    gy = gy_ref[...]
    gz = gz_ref[...]
    ix = jnp.floor((gx + 50.0) / 0.5).astype(jnp.int32)
    iy = jnp.floor((gy + 50.0) / 0.5).astype(jnp.int32)
    iz = jnp.floor((gz + 10.0) / 20.0).astype(jnp.int32)
    kept = (
        (ix >= 0) & (ix < NXY)
        & (iy >= 0) & (iy < NXY)
        & (iz >= 0) & (iz < 1)
    )
    o_ref[...] = jnp.where(kept, ix * NXY + iy, TRASH)


def _voxelize(gx2, gy2, gz2):
    return pl.pallas_call(
        _vox_body,
        grid=(pl.cdiv(VROWS, VBLK),),
        in_specs=[pl.BlockSpec((VBLK, 128), lambda i: (i, 0))] * 3,
        out_specs=pl.BlockSpec((VBLK, 128), lambda i: (i, 0)),
        out_shape=jax.ShapeDtypeStruct((VROWS, 128), jnp.int32),
    )(gx2, gy2, gz2)


KMAX = (GROUPS + NTILES - 1) // NTILES


def _sc_scatter_body(b0, vloc_hbm, x_hbm, out_hbm,
                     idx_v, rows_v, zbuf, acc, sem_ld, sem_sc, sem_z):
    c = lax.axis_index("c")
    s = lax.axis_index("s")
    col0 = c * CH

    def zrow(i, carry):
        zbuf[i, pl.ds(0, 16)] = jnp.zeros((16,), jnp.float32)
        zbuf[i, pl.ds(16, 16)] = jnp.zeros((16,), jnp.float32)
        return carry
    lax.fori_loop(0, 128, zrow, 0)

    def issue_loads(b, k, buf):
        gp = s + NTILES * k

        @pl.when(gp < GROUPS)
        def _():
            gg = (b0 + b) * GROUPS + gp
            pltpu.async_copy(
                vloc_hbm.at[pl.ds(11 * gg, 11), :], idx_v.at[buf],
                sem_ld.at[buf])
            pltpu.async_copy(
                x_hbm.at[pl.ds(704 * gg, 704), pl.ds(col0, CH)],
                rows_v.at[buf], sem_ld.at[buf])

    def wait_loads(buf):
        pltpu.make_async_copy(
            vloc_hbm.at[pl.ds(0, 11), :], idx_v.at[buf], sem_ld.at[buf]).wait()
        pltpu.make_async_copy(
            x_hbm.at[pl.ds(0, 704), pl.ds(col0, CH)], rows_v.at[buf],
            sem_ld.at[buf]).wait()

    def batch(b, carry):
        for k in range(19):
            pltpu.async_copy(
                zbuf, acc.at[pl.ds(s * ZROWS_TILE + k * 128, 128), :], sem_z)
        pltpu.async_copy(
            zbuf.at[pl.ds(0, 72), :],
            acc.at[pl.ds(s * ZROWS_TILE + 2432, 72), :], sem_z)
        for k in range(19):
            pltpu.make_async_copy(
                zbuf, acc.at[pl.ds(0, 128), :], sem_z).wait()
        pltpu.make_async_copy(
            zbuf.at[pl.ds(0, 72), :], acc.at[pl.ds(0, 72), :], sem_z).wait()
        plsc.subcore_barrier()

        issue_loads(b, 0, 0)
        for k in range(KMAX):
            cb = k % 2
            gp = s + NTILES * k
            if k >= 1:
                @pl.when(s + NTILES * (k - 1) < GROUPS)
                def _():
                    for j in range(11):
                        pltpu.make_async_copy(
                            rows_v.at[1 - cb, pl.ds(0, 64)],
                            acc.at[pl.ds(0, 64), :], sem_sc).wait()
            if k + 1 < KMAX:
                issue_loads(b, k + 1, 1 - cb)

            @pl.when(gp < GROUPS)
            def _():
                wait_loads(cb)
                for j in range(11):
                    pltpu.async_copy(
                        rows_v.at[cb, pl.ds(64 * j, 64)],
                        acc.at[idx_v.at[cb, j]], sem_sc, add=True)

        @pl.when(s + NTILES * (KMAX - 1) < GROUPS)
        def _():
            for j in range(11):
                pltpu.make_async_copy(
                    rows_v.at[(KMAX - 1) % 2, pl.ds(0, 64)],
                    acc.at[pl.ds(0, 64), :], sem_sc).wait()
        plsc.subcore_barrier()

        pltpu.sync_copy(
            acc.at[pl.ds(s * WB_TILE, WB_TILE), :],
            out_hbm.at[pl.ds(b * VOX_B + s * WB_TILE, WB_TILE),
                       pl.ds(col0, CH)])
        plsc.subcore_barrier()
        return carry

    lax.fori_loop(0, B, batch, 0)


def _sc_scatter_half(b0, vloc44, xhalf):
    nb = B
    call = functools.partial(
        pl.kernel,
        out_type=jax.ShapeDtypeStruct((nb * VOX_B, C), jnp.float32),
        mesh=plsc.VectorSubcoreMesh(core_axis_name="c", subcore_axis_name="s"),
        compiler_params=pltpu.CompilerParams(use_tc_tiling_on_sc=False),
        scratch_types=[
            pltpu.VMEM((2, 11, 64), jnp.int32),
            pltpu.VMEM((2, 704, CH), jnp.float32),
            pltpu.VMEM((128, CH), jnp.float32),
            pltpu.VMEM_SHARED((ACC_ROWS, CH), jnp.float32),
            pltpu.SemaphoreType.DMA((2,)),
            pltpu.SemaphoreType.DMA,
            pltpu.SemaphoreType.DMA,
        ],
    )(functools.partial(_sc_scatter_body, b0))
    return call(vloc44, xhalf)


def _tr_body(i_ref, o_ref):
    o_ref[0] = i_ref[0].T


def _transpose(summed, nb):
    TR = 2048
    return pl.pallas_call(
        _tr_body,
        grid=(nb, pl.cdiv(VOX_B, TR)),
        in_specs=[pl.BlockSpec((1, TR, C), lambda b, r: (b, r, 0))],
        out_specs=pl.BlockSpec((1, C, TR), lambda b, r: (b, 0, r)),
        out_shape=jax.ShapeDtypeStruct((nb, C, VOX_B), jnp.float32),
    )(summed.reshape(nb, VOX_B, C))


def kernel(x, geom):
    xp = jnp.transpose(x, (0, 1, 2, 4, 3, 5)).reshape(P_TOTAL, C)
    gx2 = geom[..., 0].reshape(VROWS, 128)
    gy2 = geom[..., 1].reshape(VROWS, 128)
    gz2 = geom[..., 2].reshape(VROWS, 128)
    vloc_hw = _voxelize(gx2, gy2, gz2).reshape(B, N, D, H, W)
    vloc64 = jnp.transpose(vloc_hw, (0, 1, 2, 4, 3)).reshape(P_TOTAL // 64, 64)
    summed = _sc_scatter_half(0, vloc64, xp)
    out = jnp.swapaxes(summed.reshape(B, VOX_B, C), 1, 2)
    return out.reshape(B, C, NXY, NXY)
